# baseline (speedup 1.0000x reference)
import jax
import jax.numpy as jnp
from jax import lax
from jax.experimental import pallas as pl
from jax.experimental.pallas import tpu as pltpu
from jax.experimental.pallas import tpu_sc as plsc

ROWS = 1_000_000
COLS = 32
NC = 2
NS = 16
NW = NC * NS
R = 504
T = -(-ROWS // R)
G = -(-T // NW)
NBUF = 2


def _copy_body(x_hbm, out_hbm, slab, *rest):
    sin = rest[:NBUF]
    sout = rest[NBUF:]
    sid = lax.axis_index("s")
    wid = sid * NC + lax.axis_index("c")
    bufs = [slab.at[sid, b] for b in range(NBUF)]

    def base(i):
        t = jnp.minimum(wid * G + i, T - 1)
        return pl.multiple_of(jnp.minimum(t * R, ROWS - R), 8)

    def start_in(i):
        b = i % NBUF
        pltpu.make_async_copy(x_hbm.at[pl.ds(base(i), R)], bufs[b], sin[b]).start()

    for g in range(NBUF):
        start_in(g)
    for g in range(G):
        b = g % NBUF
        pltpu.make_async_copy(x_hbm.at[pl.ds(base(g), R)], bufs[b], sin[b]).wait()
        pltpu.make_async_copy(bufs[b], out_hbm.at[pl.ds(base(g), R)], sout[b]).start()
        j = g - 1
        if j >= 0 and j + NBUF < G:
            jb = j % NBUF
            pltpu.make_async_copy(bufs[jb], out_hbm.at[pl.ds(base(j), R)], sout[jb]).wait()
            start_in(j + NBUF)
    for j in range(max(0, G - NBUF), G):
        jb = j % NBUF
        pltpu.make_async_copy(bufs[jb], out_hbm.at[pl.ds(base(j), R)], sout[jb]).wait()


@jax.jit
def kernel(x):
    return pl.kernel(
        _copy_body,
        out_type=jax.ShapeDtypeStruct((ROWS, COLS), jnp.float32),
        mesh=plsc.VectorSubcoreMesh(core_axis_name="c", subcore_axis_name="s"),
        compiler_params=pltpu.CompilerParams(use_tc_tiling_on_sc=False),
        scratch_types=(
            [pltpu.VMEM_SHARED((NS, NBUF, R, COLS), jnp.float32)]
            + [pltpu.SemaphoreType.DMA for _ in range(2 * NBUF)]
        ),
    )(x)


# speedup vs baseline: 1.2083x; 1.2083x over previous
import jax
import jax.numpy as jnp
from jax import lax
from jax.experimental import pallas as pl
from jax.experimental.pallas import tpu as pltpu
from jax.experimental.pallas import tpu_sc as plsc

ROWS = 1_000_000
COLS = 32
NC = 2
NS = 16
NW = NC * NS
R = 504
T = -(-ROWS // R)
G = -(-T // NW)
NBUF = 2


def _copy_body(x_hbm, out_hbm, slab, *rest):
    sin = rest[:NBUF]
    sout = rest[NBUF:]
    sid = lax.axis_index("s")
    wid = sid * NC + lax.axis_index("c")
    bufs = [slab.at[sid, b] for b in range(NBUF)]

    def base(i):
        t = jnp.minimum(wid * G + i, T - 1)
        return pl.multiple_of(jnp.minimum(t * R, ROWS - R), 8)

    def start_in(i):
        b = i % NBUF
        pltpu.make_async_copy(x_hbm.at[pl.ds(base(i), R)], bufs[b], sin[b]).start()

    for g in range(NBUF):
        start_in(g)
    for g in range(G):
        b = g % NBUF
        pltpu.make_async_copy(x_hbm.at[pl.ds(base(g), R)], bufs[b], sin[b]).wait()
        pltpu.make_async_copy(bufs[b], out_hbm.at[pl.ds(base(g), R)], sout[b]).start()
        j = g - 1
        if j >= 0 and j + NBUF < G:
            jb = j % NBUF
            pltpu.make_async_copy(bufs[jb], out_hbm.at[pl.ds(base(j), R)], sout[jb]).wait()
            start_in(j + NBUF)
    for j in range(max(0, G - NBUF), G):
        jb = j % NBUF
        pltpu.make_async_copy(bufs[jb], out_hbm.at[pl.ds(base(j), R)], sout[jb]).wait()


@jax.jit
def kernel(x):
    return pl.kernel(
        _copy_body,
        out_type=jax.ShapeDtypeStruct((ROWS, COLS), jnp.float32),
        mesh=plsc.VectorSubcoreMesh(core_axis_name="c", subcore_axis_name="s"),
        scratch_types=(
            [pltpu.VMEM_SHARED((NS, NBUF, R, COLS), jnp.float32)]
            + [pltpu.SemaphoreType.DMA for _ in range(2 * NBUF)]
        ),
    )(x)
